# baseline (speedup 1.0000x reference)
import functools

import jax
import jax.numpy as jnp
from jax import lax
from jax.experimental import pallas as pl
from jax.experimental.pallas import tpu as pltpu
from jax.experimental.pallas import tpu_sc as plsc

_K = 64
_N = 50000
_D = 256
_DP = _D + 1
_T = 80
_NCHUNK = _N // _T
_NW = 32
_SLOTS = -(-_NCHUNK // _NW)
_UNROLL = 32

_mesh = plsc.VectorSubcoreMesh(core_axis_name="c", subcore_axis_name="s")


@functools.partial(
    pl.kernel,
    out_type=jax.ShapeDtypeStruct((_NW, 16), jnp.float32),
    mesh=_mesh,
    scratch_types=[
        pltpu.VMEM((_T, _DP), jnp.float32),
        pltpu.VMEM((_T, _DP), jnp.float32),
        pltpu.VMEM((_T,), jnp.int32),
        pltpu.VMEM((_T,), jnp.int32),
        pltpu.VMEM((_K, _DP), jnp.float32),
        pltpu.VMEM((_K, _DP), jnp.float32),
        pltpu.VMEM((_K,), jnp.float32),
        pltpu.VMEM((16,), jnp.float32),
        pltpu.SemaphoreType.DMA,
        pltpu.SemaphoreType.DMA,
        pltpu.SemaphoreType.DMA,
        pltpu.SemaphoreType.DMA,
    ],
    compiler_params=pltpu.CompilerParams(
        use_tc_tiling_on_sc=False, needs_layout_passes=False),
)
def _sc_loss(x_hbm, a_hbm, c_hbm, out_hbm,
             x_v0, x_v1, a_v0, a_v1, c_v, s_v, cnt_v, p_v,
             sx0, sx1, sa0, sa1):
    wid = lax.axis_index("s") * 2 + lax.axis_index("c")
    pltpu.sync_copy(c_hbm, c_v.at[:, pl.ds(0, _D)])

    lanes = lax.broadcasted_iota(jnp.int32, (16,), 0)
    ones = jnp.ones((16,), jnp.int32)
    fzeros = jnp.zeros((16,), jnp.float32)
    fones = jnp.ones((16,), jnp.float32)
    bufs = ((x_v0, a_v0, sx0, sa0), (x_v1, a_v1, sx1, sa1))

    def zrow(i, _):
        for k in range(_D // 16):
            s_v[i, pl.ds(k * 16, 16)] = fzeros
        return 0
    lax.fori_loop(0, _K, zrow, 0)
    for k in range(_K // 16):
        cnt_v[pl.ds(k * 16, 16)] = fzeros

    def start(slot, buf):
        x_v, a_v, sx, sa = buf
        off = jnp.minimum(wid + slot * _NW, _NCHUNK - 1) * _T
        pltpu.make_async_copy(
            x_hbm.at[pl.ds(off, _T), :], x_v.at[:, pl.ds(0, _D)], sx).start()
        pltpu.make_async_copy(a_hbm.at[pl.ds(off, _T)], a_v, sa).start()

    def process(slot, buf, acc):
        x_v, a_v, sx, sa = buf
        pltpu.make_async_copy(
            x_hbm.at[pl.ds(0, _T), :], x_v.at[:, pl.ds(0, _D)], sx).wait()
        pltpu.make_async_copy(a_hbm.at[pl.ds(0, _T)], a_v, sa).wait()
        valid = (wid + slot * _NW) < _NCHUNK
        vmask = jnp.where(valid, fones, fzeros)
        vbool = vmask > 0.5
        for g in range(_T // 16):
            pvec = lanes + (g * 16)
            va = a_v[pl.ds(g * 16, 16)]
            plsc.addupdate_scatter(cnt_v, [va], vmask)

            def dim_blk(b, carry):
                dvec, p_in = carry
                for _ in range(_UNROLL):
                    vx = plsc.load_gather(x_v, [pvec, dvec])
                    plsc.addupdate_scatter(s_v, [va, dvec], vx, mask=vbool)
                    p_in = p_in + vx * vx
                    dvec = dvec + ones
                return dvec, p_in

            _, part = lax.fori_loop(0, _D // _UNROLL, dim_blk,
                                    (jnp.zeros((16,), jnp.int32), fzeros))
            acc = acc + vmask * part
        return acc

    start(0, bufs[0])

    def slot_pair(t, acc):
        s0 = t * 2
        start(s0 + 1, bufs[1])
        acc = process(s0, bufs[0], acc)
        start(s0 + 2, bufs[0])
        acc = process(s0 + 1, bufs[1], acc)
        return acc

    nacc = lax.fori_loop(0, _SLOTS // 2, slot_pair, fzeros)
    pltpu.make_async_copy(
        x_hbm.at[pl.ds(0, _T), :], x_v0.at[:, pl.ds(0, _D)], sx0).wait()
    pltpu.make_async_copy(a_hbm.at[pl.ds(0, _T)], a_v0, sa0).wait()

    def crow(i, carry):
        cross, cw = carry
        cni = fzeros
        for k in range(_D // 16):
            vc = c_v[i, pl.ds(k * 16, 16)]
            vs = s_v[i, pl.ds(k * 16, 16)]
            cross = cross + vc * vs
            cni = cni + vc * vc
        cnt_i = plsc.load_gather(cnt_v, [jnp.full((16,), i, jnp.int32)])
        return cross, cw + cnt_i * cni
    cross, cw = lax.fori_loop(0, _K, crow, (fzeros, fzeros))

    p_v[...] = nacc - 2.0 * cross + cw
    pltpu.sync_copy(p_v, out_hbm.at[wid])


def kernel(x, cluster_assignments, cluster_centers):
    partials = _sc_loss(x, cluster_assignments, cluster_centers)
    return jnp.sum(partials)


# speedup vs baseline: 2.4344x; 2.4344x over previous
import functools

import jax
import jax.numpy as jnp
from jax import lax
from jax.experimental import pallas as pl
from jax.experimental.pallas import tpu as pltpu
from jax.experimental.pallas import tpu_sc as plsc

_K = 64
_N = 50000
_D = 256
_T = 80
_NCHUNK = _N // _T
_NW = 32
_SLOTS = -(-_NCHUNK // _NW)
_UNROLL = 32

_mesh = plsc.VectorSubcoreMesh(core_axis_name="c", subcore_axis_name="s")


@functools.partial(
    pl.kernel,
    out_type=jax.ShapeDtypeStruct((_NW, 16), jnp.float32),
    mesh=_mesh,
    scratch_types=[
        pltpu.VMEM((_T * _D,), jnp.float32),
        pltpu.VMEM((_T * _D,), jnp.float32),
        pltpu.VMEM((_T,), jnp.int32),
        pltpu.VMEM((_T,), jnp.int32),
        pltpu.VMEM((_K * _D,), jnp.float32),
        pltpu.VMEM((16,), jnp.float32),
        pltpu.SemaphoreType.DMA,
        pltpu.SemaphoreType.DMA,
        pltpu.SemaphoreType.DMA,
        pltpu.SemaphoreType.DMA,
    ],
    compiler_params=pltpu.CompilerParams(
        use_tc_tiling_on_sc=False, needs_layout_passes=False),
)
def _sc_loss(x_hbm, a_hbm, c_hbm, out_hbm,
             x_v0, x_v1, a_v0, a_v1, c_v, p_v,
             sx0, sx1, sa0, sa1):
    wid = lax.axis_index("s") * 2 + lax.axis_index("c")
    pltpu.sync_copy(c_hbm, c_v)

    lanes = lax.broadcasted_iota(jnp.int32, (16,), 0)
    ones = jnp.ones((16,), jnp.int32)
    dmask = jnp.full((16,), _D - 1, jnp.int32)
    fzeros = jnp.zeros((16,), jnp.float32)
    bufs = ((x_v0, a_v0, sx0, sa0), (x_v1, a_v1, sx1, sa1))

    def start(slot, buf):
        x_v, a_v, sx, sa = buf
        off = jnp.minimum(wid + slot * _NW, _NCHUNK - 1) * _T
        pltpu.make_async_copy(
            x_hbm.at[pl.ds(off * _D, _T * _D)], x_v, sx).start()
        pltpu.make_async_copy(a_hbm.at[pl.ds(off, _T)], a_v, sa).start()

    def process(slot, buf, acc):
        x_v, a_v, sx, sa = buf
        pltpu.make_async_copy(
            x_hbm.at[pl.ds(0, _T * _D)], x_v, sx).wait()
        pltpu.make_async_copy(a_hbm.at[pl.ds(0, _T)], a_v, sa).wait()
        valid = (wid + slot * _NW) < _NCHUNK
        vmask = jnp.where(valid, jnp.ones((16,), jnp.float32), fzeros)
        for g in range(_T // 16):
            xbase = (lanes + (g * 16)) * _D
            cbase = a_v[pl.ds(g * 16, 16)] * _D

            def dim_blk(b, carry):
                dl, p_in = carry
                for _ in range(_UNROLL):
                    vx = plsc.load_gather(x_v, [xbase + dl])
                    vc = plsc.load_gather(c_v, [cbase + dl])
                    diff = vx - vc
                    p_in = p_in + diff * diff
                    dl = (dl + ones) & dmask
                return dl, p_in

            _, part = lax.fori_loop(0, _D // _UNROLL, dim_blk,
                                    (lanes, fzeros))
            acc = acc + vmask * part
        return acc

    start(0, bufs[0])

    def slot_pair(t, acc):
        s0 = t * 2
        start(s0 + 1, bufs[1])
        acc = process(s0, bufs[0], acc)
        start(s0 + 2, bufs[0])
        acc = process(s0 + 1, bufs[1], acc)
        return acc

    acc = lax.fori_loop(0, _SLOTS // 2, slot_pair, fzeros)
    pltpu.make_async_copy(x_hbm.at[pl.ds(0, _T * _D)], x_v0, sx0).wait()
    pltpu.make_async_copy(a_hbm.at[pl.ds(0, _T)], a_v0, sa0).wait()

    p_v[...] = acc
    pltpu.sync_copy(p_v, out_hbm.at[wid])


def kernel(x, cluster_assignments, cluster_centers):
    partials = _sc_loss(x.reshape(-1), cluster_assignments,
                        cluster_centers.reshape(-1))
    return jnp.sum(partials)


# speedup vs baseline: 2.5840x; 1.0615x over previous
import functools

import jax
import jax.numpy as jnp
from jax import lax
from jax.experimental import pallas as pl
from jax.experimental.pallas import tpu as pltpu
from jax.experimental.pallas import tpu_sc as plsc

_K = 64
_N = 50000
_D = 256
_T = 80
_NCHUNK = _N // _T
_NW = 32
_SLOTS = -(-_NCHUNK // _NW)
_UNROLL = 32

_mesh = plsc.VectorSubcoreMesh(core_axis_name="c", subcore_axis_name="s")


@functools.partial(
    pl.kernel,
    out_type=jax.ShapeDtypeStruct((_NW, 16), jnp.float32),
    mesh=_mesh,
    scratch_types=[
        pltpu.VMEM((_T * _D,), jnp.float32),
        pltpu.VMEM((_T * _D,), jnp.float32),
        pltpu.VMEM((_T,), jnp.int32),
        pltpu.VMEM((_T,), jnp.int32),
        pltpu.VMEM((_K * _D,), jnp.float32),
        pltpu.VMEM((16,), jnp.float32),
        pltpu.SemaphoreType.DMA,
        pltpu.SemaphoreType.DMA,
        pltpu.SemaphoreType.DMA,
        pltpu.SemaphoreType.DMA,
    ],
    compiler_params=pltpu.CompilerParams(
        use_tc_tiling_on_sc=False, needs_layout_passes=False),
)
def _sc_loss(x_hbm, a_hbm, c_hbm, out_hbm,
             x_v0, x_v1, a_v0, a_v1, c_v, p_v,
             sx0, sx1, sa0, sa1):
    wid = lax.axis_index("s") * 2 + lax.axis_index("c")
    pltpu.sync_copy(c_hbm, c_v)

    lanes = lax.broadcasted_iota(jnp.int32, (16,), 0)
    ones = jnp.ones((16,), jnp.int32)
    dmask = jnp.full((16,), _D - 1, jnp.int32)
    fzeros = jnp.zeros((16,), jnp.float32)
    bufs = ((x_v0, a_v0, sx0, sa0), (x_v1, a_v1, sx1, sa1))

    def start(slot, buf):
        x_v, a_v, sx, sa = buf
        off = jnp.minimum(wid + slot * _NW, _NCHUNK - 1) * _T
        pltpu.make_async_copy(
            x_hbm.at[pl.ds(off * _D, _T * _D)], x_v, sx).start()
        pltpu.make_async_copy(a_hbm.at[pl.ds(off, _T)], a_v, sa).start()

    def process(slot, buf, acc):
        x_v, a_v, sx, sa = buf
        pltpu.make_async_copy(
            x_hbm.at[pl.ds(0, _T * _D)], x_v, sx).wait()
        pltpu.make_async_copy(a_hbm.at[pl.ds(0, _T)], a_v, sa).wait()
        valid = (wid + slot * _NW) < _NCHUNK
        vmask = jnp.where(valid, jnp.ones((16,), jnp.float32), fzeros)
        for g in range(_T // 16):
            xbase = (lanes + (g * 16)) * _D
            cbase = a_v[pl.ds(g * 16, 16)] * _D

            def dim_blk(b, carry):
                dl0, p0, p1, p2, p3 = carry
                accs = [p0, p1, p2, p3]
                for u in range(_UNROLL):
                    dlu = (dl0 + u) & dmask
                    vx = plsc.load_gather(x_v, [xbase + dlu])
                    vc = plsc.load_gather(c_v, [cbase + dlu])
                    diff = vx - vc
                    accs[u % 4] = accs[u % 4] + diff * diff
                dl0 = (dl0 + _UNROLL) & dmask
                return (dl0, accs[0], accs[1], accs[2], accs[3])

            _, p0, p1, p2, p3 = lax.fori_loop(
                0, _D // _UNROLL, dim_blk,
                (lanes, fzeros, fzeros, fzeros, fzeros))
            acc = acc + vmask * ((p0 + p1) + (p2 + p3))
        return acc

    start(0, bufs[0])

    def slot_pair(t, acc):
        s0 = t * 2
        start(s0 + 1, bufs[1])
        acc = process(s0, bufs[0], acc)
        start(s0 + 2, bufs[0])
        acc = process(s0 + 1, bufs[1], acc)
        return acc

    acc = lax.fori_loop(0, _SLOTS // 2, slot_pair, fzeros)
    pltpu.make_async_copy(x_hbm.at[pl.ds(0, _T * _D)], x_v0, sx0).wait()
    pltpu.make_async_copy(a_hbm.at[pl.ds(0, _T)], a_v0, sa0).wait()

    p_v[...] = acc
    pltpu.sync_copy(p_v, out_hbm.at[wid])


def kernel(x, cluster_assignments, cluster_centers):
    partials = _sc_loss(x.reshape(-1), cluster_assignments,
                        cluster_centers.reshape(-1))
    return jnp.sum(partials)
